# baseline (speedup 1.0000x reference)
import jax
import jax.numpy as jnp
from jax import lax
from jax.experimental import pallas as pl
from jax.experimental.pallas import tpu as pltpu
from jax.experimental.pallas import tpu_sc as plsc

NN = 100000
NPAD = 100096
HD = 16
NE = 1600000
NT = 16
B = 256
IDXW = 128
NSUB = B // IDXW
EPT = 101120
EPAD = NT * EPT
NCHUNK = EPT // B
NCHT = EPAD // B
NSLOT = 5
ZR = NPAD // NT
ZFULL = ZR // B
ZREM = ZR - ZFULL * B

_mesh = plsc.VectorSubcoreMesh(core_axis_name="c", subcore_axis_name="s")


def _layer_body(emb, sdv, out, sdvx, rows, acc, semz, semi0, semi1, semi2,
                semi3, semi4, semg0, semg1, semg2, semg3, semg4, sems0,
                sems1, sems2, sems3, sems4):
    c = lax.axis_index("c")
    s = lax.axis_index("s")
    semi = (semi0, semi1, semi2, semi3, semi4)
    semg = (semg0, semg1, semg2, semg3, semg4)
    sems = (sems0, sems1, sems2, sems3, sems4)

    @pl.loop(0, B)
    def _(i):
        rows[0, i] = jnp.zeros((HD,), jnp.float32)

    zbase = s * ZR

    def zero_cps():
        cps = [pltpu.make_async_copy(rows.at[0],
                                     acc.at[pl.ds(zbase + z * B, B)], semz)
               for z in range(ZFULL)]
        cps.append(pltpu.make_async_copy(rows.at[0, pl.ds(0, ZREM)],
                                         acc.at[pl.ds(zbase + ZFULL * B,
                                                      ZREM)], semz))
        return cps

    for cp in zero_cps():
        cp.start()
    for cp in zero_cps():
        cp.wait()
    plsc.subcore_barrier()

    cbase = s * NCHUNK

    def idx_cp(g, k):
        return pltpu.make_async_copy(sdv.at[c, cbase + g], sdvx.at[k],
                                     semi[k])

    def gather_cps(k):
        return [
            pltpu.make_async_copy(emb.at[sdvx.at[k, 0, j]],
                                  rows.at[k, pl.ds(j * IDXW, IDXW)], semg[k])
            for j in range(NSUB)
        ]

    def scatter_cps(k):
        return [
            pltpu.make_async_copy(rows.at[k, pl.ds(j * IDXW, IDXW)],
                                  acc.at[sdvx.at[k, 1, j]], sems[k])
            for j in range(NSUB)
        ]

    def scale(k):
        for j in range(NSUB):
            @plsc.parallel_loop(0, IDXW // 16, unroll=2)
            def _(q):
                v16 = plsc.bitcast(sdvx[k, 2, j, pl.ds(q * 16, 16)],
                                   jnp.float32)
                for i in range(16):
                    e = j * IDXW + q * 16 + i
                    rows[k, e] = rows[k, e] * v16[i]

    def half(g, k, wait_sc, fire_idx, fire_gat):
        k2, k3 = (k + 2) % NSLOT, (k + 3) % NSLOT
        if wait_sc:
            for cp in scatter_cps((k - 2) % NSLOT):
                cp.wait()
        if fire_idx:
            idx_cp(g + 3, k3).start()
        if fire_gat:
            idx_cp(g + 2, k2).wait()
            for cp in gather_cps(k2):
                cp.start()
        for cp in gather_cps(k):
            cp.wait()
        scale(k)
        for cp in scatter_cps(k):
            cp.start(add=True)

    idx_cp(0, 0).start()
    idx_cp(1, 1).start()
    idx_cp(2, 2).start()
    idx_cp(0, 0).wait()
    for cp in gather_cps(0):
        cp.start()
    idx_cp(1, 1).wait()
    for cp in gather_cps(1):
        cp.start()
    half(0, 0, False, True, True)
    half(1, 1, False, True, True)
    half(2, 2, True, True, True)
    half(3, 3, True, True, True)
    half(4, 4, True, True, True)

    @pl.loop(1, NCHUNK // NSLOT - 1)
    def _(t):
        for k in range(NSLOT):
            half(NSLOT * t + k, k, True, True, True)

    half(NCHUNK - 5, 0, True, True, True)
    half(NCHUNK - 4, 1, True, True, True)
    half(NCHUNK - 3, 2, True, False, True)
    half(NCHUNK - 2, 3, True, False, False)
    half(NCHUNK - 1, 4, True, False, False)
    for cp in scatter_cps(3):
        cp.wait()
    for cp in scatter_cps(4):
        cp.wait()
    plsc.subcore_barrier()

    ob = c * NPAD + s * ZR

    def wb_cps():
        cps = [pltpu.make_async_copy(acc.at[pl.ds(zbase + z * B, B)],
                                     out.at[pl.ds(ob + z * B, B)], semz)
               for z in range(ZFULL)]
        cps.append(pltpu.make_async_copy(
            acc.at[pl.ds(zbase + ZFULL * B, ZREM)],
            out.at[pl.ds(ob + ZFULL * B, ZREM)], semz))
        return cps

    for cp in wb_cps():
        cp.start()
    for cp in wb_cps():
        cp.wait()


_layer = pl.kernel(
    _layer_body,
    out_type=jax.ShapeDtypeStruct((2 * NPAD, HD), jnp.float32),
    mesh=_mesh,
    compiler_params=(
        pltpu.CompilerParams(use_tc_tiling_on_sc=False,
                             needs_layout_passes=False)
        if "needs_layout_passes" in pltpu.CompilerParams.__dataclass_fields__
        else pltpu.CompilerParams(use_tc_tiling_on_sc=False)),
    scratch_types=[
        pltpu.VMEM((NSLOT, 3, NSUB, IDXW), jnp.int32),
        pltpu.VMEM((NSLOT, B, HD), jnp.float32),
        pltpu.VMEM_SHARED((NPAD, HD), jnp.float32),
    ] + [pltpu.SemaphoreType.DMA] * 16,
)


def _mean_body(a0, b0, a1, b1, a2, b2, a3, b3, o):
    sl = (a0[0] + a1[0] + a2[0] + a3[0]) * 0.25
    sr = (b0[0] + b1[0] + b2[0] + b3[0]) * 0.25
    o[...] = jnp.concatenate([sl, sr], axis=1)


def _mean4(e0, e1, e2, e3):
    bn = 4000
    r = lambda x: x.reshape(2, NPAD, HD)
    in_l = pl.BlockSpec((1, bn, HD), lambda i: (0, i, 0))
    in_r = pl.BlockSpec((1, bn, HD), lambda i: (1, i, 0))
    call = pl.pallas_call(
        _mean_body,
        grid=(NN // bn,),
        in_specs=[in_l, in_r] * 4,
        out_specs=pl.BlockSpec((bn, 2 * HD), lambda i: (i, 0)),
        out_shape=jax.ShapeDtypeStruct((NN, 2 * HD), jnp.float32),
    )
    return call(r(e0), r(e0), r(e1), r(e1), r(e2), r(e2), r(e3), r(e3))


def kernel(user_emb, item_emb, adj_indices, adj_values):
    n_users = user_emb.shape[0]
    dst = adj_indices[0].astype(jnp.int32)
    src = adj_indices[1].astype(jnp.int32)
    val = adj_values.astype(jnp.float32)

    pad = EPAD - NE
    pad_idx = (jnp.arange(pad, dtype=jnp.int32) * 17) % NN
    src_p = jnp.concatenate([src, pad_idx])
    dst_p = jnp.concatenate([dst, pad_idx])
    val_p = jnp.concatenate([val, jnp.zeros((pad,), jnp.float32)])

    valr = lax.bitcast_convert_type(val_p, jnp.int32)
    blk = lambda x: x.reshape(NCHT, 1, NSUB, IDXW)
    pack = lambda sc_src: jnp.concatenate(
        [blk(sc_src), blk(dst_p), blk(valr)], axis=1)
    sdv = jnp.stack([pack(src_p), pack(src_p + NPAD)], axis=0)

    all_emb = jnp.concatenate([user_emb, item_emb], axis=0)
    row_pad = ((0, NPAD - NN), (0, 0))
    e0 = jnp.concatenate([jnp.pad(all_emb[:, :HD], row_pad),
                          jnp.pad(all_emb[:, HD:], row_pad)], axis=0)

    e1 = _layer(e0, sdv)
    e2 = _layer(e1, sdv)
    e3 = _layer(e2, sdv)

    final = _mean4(e0, e1, e2, e3)
    return final[:n_users], final[n_users:]


# speedup vs baseline: 1.2101x; 1.2101x over previous
import jax
import jax.numpy as jnp
from jax import lax
from jax.experimental import pallas as pl
from jax.experimental.pallas import tpu as pltpu
from jax.experimental.pallas import tpu_sc as plsc

NN = 100000
NPAD = 100096
HD = 16
NE = 1600000
NT = 16
B = 256
IDXW = 128
NSUB = B // IDXW
EPT = 101120
EPAD = NT * EPT
NCHUNK = EPT // B
NCHT = EPAD // B
NSLOT = 5
ZR = NPAD // NT
ZFULL = ZR // B
ZREM = ZR - ZFULL * B

_mesh = plsc.VectorSubcoreMesh(core_axis_name="c", subcore_axis_name="s")


def _layer_body(emb, sdv, out, sdvx, rows, acc, semz, semi0, semi1, semi2,
                semi3, semi4, semg0, semg1, semg2, semg3, semg4, sems0,
                sems1, sems2, sems3, sems4):
    c = lax.axis_index("c")
    s = lax.axis_index("s")
    semi = (semi0, semi1, semi2, semi3, semi4)
    semg = (semg0, semg1, semg2, semg3, semg4)
    sems = (sems0, sems1, sems2, sems3, sems4)

    @pl.loop(0, B)
    def _(i):
        rows[0, i] = jnp.zeros((HD,), jnp.float32)

    zbase = s * ZR

    def zero_cps():
        cps = [pltpu.make_async_copy(rows.at[0],
                                     acc.at[pl.ds(zbase + z * B, B)], semz)
               for z in range(ZFULL)]
        cps.append(pltpu.make_async_copy(rows.at[0, pl.ds(0, ZREM)],
                                         acc.at[pl.ds(zbase + ZFULL * B,
                                                      ZREM)], semz))
        return cps

    for cp in zero_cps():
        cp.start()
    for cp in zero_cps():
        cp.wait()
    plsc.subcore_barrier()

    cbase = s * NCHUNK

    def idx_cp(g, k):
        return pltpu.make_async_copy(sdv.at[c, cbase + g], sdvx.at[k],
                                     semi[k])

    def gather_cps(k):
        return [
            pltpu.make_async_copy(emb.at[sdvx.at[k, 0, j]],
                                  rows.at[k, pl.ds(j * IDXW, IDXW)], semg[k])
            for j in range(NSUB)
        ]

    def scatter_cps(k):
        return [
            pltpu.make_async_copy(rows.at[k, pl.ds(j * IDXW, IDXW)],
                                  acc.at[sdvx.at[k, 1, j]], sems[k])
            for j in range(NSUB)
        ]

    def scale(k):
        for j in range(NSUB):
            @plsc.parallel_loop(0, IDXW // 16, unroll=2)
            def _(q):
                v16 = plsc.bitcast(sdvx[k, 2, j, pl.ds(q * 16, 16)],
                                   jnp.float32)
                for i in range(16):
                    e = j * IDXW + q * 16 + i
                    rows[k, e] = rows[k, e] * v16[i]

    def half(g, k, wait_sc, fire_idx, fire_gat):
        k2, k3 = (k + 2) % NSLOT, (k + 3) % NSLOT
        if wait_sc:
            for cp in scatter_cps((k - 2) % NSLOT):
                cp.wait()
        if fire_idx:
            idx_cp(g + 3, k3).start()
        if fire_gat:
            idx_cp(g + 2, k2).wait()
            for cp in gather_cps(k2):
                cp.start()
        for cp in gather_cps(k):
            cp.wait()
        scale(k)
        for cp in scatter_cps(k):
            cp.start(add=True)

    idx_cp(0, 0).start()
    idx_cp(1, 1).start()
    idx_cp(2, 2).start()
    idx_cp(0, 0).wait()
    for cp in gather_cps(0):
        cp.start()
    idx_cp(1, 1).wait()
    for cp in gather_cps(1):
        cp.start()
    half(0, 0, False, True, True)
    half(1, 1, False, True, True)
    half(2, 2, True, True, True)
    half(3, 3, True, True, True)
    half(4, 4, True, True, True)

    @pl.loop(1, NCHUNK // NSLOT - 1)
    def _(t):
        for k in range(NSLOT):
            half(NSLOT * t + k, k, True, True, True)

    half(NCHUNK - 5, 0, True, True, True)
    half(NCHUNK - 4, 1, True, True, True)
    half(NCHUNK - 3, 2, True, False, True)
    half(NCHUNK - 2, 3, True, False, False)
    half(NCHUNK - 1, 4, True, False, False)
    for cp in scatter_cps(3):
        cp.wait()
    for cp in scatter_cps(4):
        cp.wait()
    plsc.subcore_barrier()

    ob = c * NPAD + s * ZR

    def wb_cps():
        cps = [pltpu.make_async_copy(acc.at[pl.ds(zbase + z * B, B)],
                                     out.at[pl.ds(ob + z * B, B)], semz)
               for z in range(ZFULL)]
        cps.append(pltpu.make_async_copy(
            acc.at[pl.ds(zbase + ZFULL * B, ZREM)],
            out.at[pl.ds(ob + ZFULL * B, ZREM)], semz))
        return cps

    for cp in wb_cps():
        cp.start()
    for cp in wb_cps():
        cp.wait()


_layer = pl.kernel(
    _layer_body,
    out_type=jax.ShapeDtypeStruct((2 * NPAD, HD), jnp.float32),
    mesh=_mesh,
    compiler_params=(
        pltpu.CompilerParams(use_tc_tiling_on_sc=False,
                             needs_layout_passes=False)
        if "needs_layout_passes" in pltpu.CompilerParams.__dataclass_fields__
        else pltpu.CompilerParams(use_tc_tiling_on_sc=False)),
    scratch_types=[
        pltpu.VMEM((NSLOT, 3, NSUB, IDXW), jnp.int32),
        pltpu.VMEM((NSLOT, B, HD), jnp.float32),
        pltpu.VMEM_SHARED((NPAD, HD), jnp.float32),
    ] + [pltpu.SemaphoreType.DMA] * 16,
)


RM = 625
NUSER = 60000
NITEM = 40000


def _mean_body(e0, e1, e2, e3, uo, io, buf, obuf, sem):
    c = lax.axis_index("c")
    s = lax.axis_index("s")
    w = c * NT + s

    def seg(out_ref, nbase, rows_per_w, nchunks):
        @pl.loop(0, nchunks)
        def _(z):
            ro = w * rows_per_w + z * RM
            r0 = nbase + ro
            cps = []
            for a, e in enumerate((e0, e1, e2, e3)):
                cps.append(pltpu.make_async_copy(
                    e.at[pl.ds(r0, RM)], buf.at[2 * a], sem))
                cps.append(pltpu.make_async_copy(
                    e.at[pl.ds(NPAD + r0, RM)], buf.at[2 * a + 1], sem))
            for cp in cps:
                cp.start()
            for cp in cps:
                cp.wait()

            @plsc.parallel_loop(0, RM, unroll=2)
            def _(r):
                lhs = (buf[0, r] + buf[2, r] + buf[4, r] + buf[6, r]) * 0.25
                rhs = (buf[1, r] + buf[3, r] + buf[5, r] + buf[7, r]) * 0.25
                obuf[r, pl.ds(0, HD)] = lhs
                obuf[r, pl.ds(HD, HD)] = rhs

            pltpu.sync_copy(obuf, out_ref.at[pl.ds(ro, RM)])

    seg(uo, 0, NUSER // (2 * NT), NUSER // (2 * NT) // RM)
    seg(io, NUSER, NITEM // (2 * NT), NITEM // (2 * NT) // RM)


_mean4 = pl.kernel(
    _mean_body,
    out_type=(jax.ShapeDtypeStruct((NUSER, 2 * HD), jnp.float32),
              jax.ShapeDtypeStruct((NITEM, 2 * HD), jnp.float32)),
    mesh=_mesh,
    compiler_params=(
        pltpu.CompilerParams(use_tc_tiling_on_sc=False,
                             needs_layout_passes=False)
        if "needs_layout_passes" in pltpu.CompilerParams.__dataclass_fields__
        else pltpu.CompilerParams(use_tc_tiling_on_sc=False)),
    scratch_types=[
        pltpu.VMEM((8, RM, HD), jnp.float32),
        pltpu.VMEM((RM, 2 * HD), jnp.float32),
        pltpu.SemaphoreType.DMA,
    ],
)


def kernel(user_emb, item_emb, adj_indices, adj_values):
    n_users = user_emb.shape[0]
    dst = adj_indices[0].astype(jnp.int32)
    src = adj_indices[1].astype(jnp.int32)
    val = adj_values.astype(jnp.float32)

    pad = EPAD - NE
    pad_idx = (jnp.arange(pad, dtype=jnp.int32) * 17) % NN
    src_p = jnp.concatenate([src, pad_idx])
    dst_p = jnp.concatenate([dst, pad_idx])
    val_p = jnp.concatenate([val, jnp.zeros((pad,), jnp.float32)])

    valr = lax.bitcast_convert_type(val_p, jnp.int32)
    blk = lambda x: x.reshape(NCHT, 1, NSUB, IDXW)
    pack = lambda sc_src: jnp.concatenate(
        [blk(sc_src), blk(dst_p), blk(valr)], axis=1)
    sdv = jnp.stack([pack(src_p), pack(src_p + NPAD)], axis=0)

    all_emb = jnp.concatenate([user_emb, item_emb], axis=0)
    row_pad = ((0, NPAD - NN), (0, 0))
    e0 = jnp.concatenate([jnp.pad(all_emb[:, :HD], row_pad),
                          jnp.pad(all_emb[:, HD:], row_pad)], axis=0)

    e1 = _layer(e0, sdv)
    e2 = _layer(e1, sdv)
    e3 = _layer(e2, sdv)

    del n_users
    user_all, item_all = _mean4(e0, e1, e2, e3)
    return user_all, item_all
